# baseline (speedup 1.0000x reference)
import jax
import jax.numpy as jnp
from jax import lax
from jax.experimental import pallas as pl
from jax.experimental.pallas import tpu as pltpu
from jax.experimental.pallas import tpu_sc as plsc

MEL = 4096
TM = 512
MAXD = 8
FPW = 1024
CH = 128


def _layer_norm(h, g, b):
    mu = jnp.mean(h, axis=1, keepdims=True)
    var = jnp.mean((h - mu) ** 2, axis=1, keepdims=True)
    return (h - mu) / jnp.sqrt(var + 1e-5) * g + b


def _tc_body(x_ref, t_ref, mml_ref,
             w1p, w1c, w1n, b1, g1, be1,
             w2p, w2c, w2n, b2, g2, be2,
             lw, lb,
             al_ref, dp_ref):
    L = t_ref.shape[2]
    D = x_ref.shape[2]
    mt = pl.program_id(1)

    dur = t_ref[0].astype(jnp.float32)
    tri = (lax.broadcasted_iota(jnp.int32, (L, L), 0)
           <= lax.broadcasted_iota(jnp.int32, (L, L), 1)).astype(jnp.float32)
    csum = jnp.dot(dur, tri, preferred_element_type=jnp.float32)

    mvec = (mt * TM + lax.broadcasted_iota(jnp.int32, (TM, 1), 0)
            ).astype(jnp.float32)
    al_ref[0] = ((csum > mvec) & (mvec >= csum - dur)
                 & (mvec < mml_ref[0, 0])).astype(jnp.float32)

    @pl.when(mt == 0)
    def _dp():
        xb = x_ref[0]
        zr = jnp.zeros((1, D), jnp.float32)
        xp = jnp.concatenate([zr, xb[:-1]], axis=0)
        xn = jnp.concatenate([xb[1:], zr], axis=0)
        h = (jnp.dot(xp, w1p[...], preferred_element_type=jnp.float32)
             + jnp.dot(xb, w1c[...], preferred_element_type=jnp.float32)
             + jnp.dot(xn, w1n[...], preferred_element_type=jnp.float32)
             + b1[...])
        h = jax.nn.relu(_layer_norm(h, g1[...], be1[...]))
        hp = jnp.concatenate([zr, h[:-1]], axis=0)
        hn = jnp.concatenate([h[1:], zr], axis=0)
        h2 = (jnp.dot(hp, w2p[...], preferred_element_type=jnp.float32)
              + jnp.dot(h, w2c[...], preferred_element_type=jnp.float32)
              + jnp.dot(hn, w2n[...], preferred_element_type=jnp.float32)
              + b2[...])
        h2 = jax.nn.relu(_layer_norm(h2, g2[...], be2[...]))
        dp = jnp.exp(jnp.sum(h2 * lw[...], axis=1) + lb[0, 0])
        dp_ref[0] = dp.reshape(1, L)


def _sc_body(xflat_hbm, tgt_hbm, out_hbm, tvec, ids, rows0, rows1, sem0, sem1):
    L = tgt_hbm.shape[1]
    padl = L + MAXD
    wid = lax.axis_index("s") * 2 + lax.axis_index("c")
    wpb = MEL // FPW
    b = wid // wpb
    f0 = (wid % wpb) * FPW

    pltpu.sync_copy(tgt_hbm.at[b], tvec)

    zid = b * padl + L
    zvec = jnp.full((16,), zid, jnp.int32)
    for i in range(FPW // 16):
        ids[pl.ds(i * 16, 16)] = zvec

    lane = lax.broadcasted_iota(jnp.int32, (16,), 0)
    carry = jnp.int32(0)
    for v in range(L // 16):
        dur_v = tvec[pl.ds(v * 16, 16)]
        cs = plsc.cumsum(dur_v) + carry
        carry = carry + jnp.sum(dur_v)
        rel = cs - dur_v - f0
        tok = b * padl + v * 16 + lane
        for k in range(MAXD - 1):
            idx = rel + k
            msk = (idx >= 0) & (idx < FPW) & (dur_v > k)
            plsc.store_scatter(ids, [jnp.where(msk, idx, 0)], tok, mask=msk)

    rows = (rows0, rows1)
    sems = (sem0, sem1)
    nch = FPW // CH
    cps = [None, None]
    cps[0] = pltpu.async_copy(xflat_hbm.at[ids.at[pl.ds(0, CH)]], rows0, sem0)
    for j in range(nch):
        cps[j % 2].wait()
        if j + 1 < nch:
            cps[(j + 1) % 2] = pltpu.async_copy(
                xflat_hbm.at[ids.at[pl.ds((j + 1) * CH, CH)]],
                rows[(j + 1) % 2], sems[(j + 1) % 2])
        pltpu.sync_copy(rows[j % 2], out_hbm.at[pl.ds(wid * FPW + j * CH, CH)])


def kernel(x, target, mel_max_length,
           conv1_w, conv1_b, ln1_g, ln1_b,
           conv2_w, conv2_b, ln2_g, ln2_b,
           lin_w, lin_b):
    B, L, D = x.shape
    F = conv1_w.shape[0]
    t3 = target.reshape(B, 1, L)
    mml = jnp.asarray(mel_max_length, jnp.float32).reshape(1, 1)
    w1p = conv1_w[:, :, 0].T
    w1c = conv1_w[:, :, 1].T
    w1n = conv1_w[:, :, 2].T
    w2p = conv2_w[:, :, 0].T
    w2c = conv2_w[:, :, 1].T
    w2n = conv2_w[:, :, 2].T
    b1 = conv1_b.reshape(1, F)
    g1 = ln1_g.reshape(1, F)
    be1 = ln1_b.reshape(1, F)
    b2 = conv2_b.reshape(1, F)
    g2 = ln2_g.reshape(1, F)
    be2 = ln2_b.reshape(1, F)
    lw = lin_w.reshape(1, F)
    lb = lin_b.reshape(1, 1)

    xflat = jnp.concatenate(
        [x, jnp.zeros((B, MAXD, D), jnp.float32)], axis=1).reshape(B * (L + MAXD), D)
    mesh = plsc.VectorSubcoreMesh(core_axis_name="c", subcore_axis_name="s")
    out_flat = pl.kernel(
        _sc_body,
        out_type=jax.ShapeDtypeStruct((B * MEL, D), jnp.float32),
        mesh=mesh,
        compiler_params=pltpu.CompilerParams(needs_layout_passes=False),
        scratch_types=[
            pltpu.VMEM((L,), jnp.int32),
            pltpu.VMEM((FPW,), jnp.int32),
            pltpu.VMEM((CH, D), jnp.float32),
            pltpu.VMEM((CH, D), jnp.float32),
            pltpu.SemaphoreType.DMA,
            pltpu.SemaphoreType.DMA,
        ],
    )(xflat, target)

    const = lambda *_: (0, 0)
    align, dp3 = pl.pallas_call(
        _tc_body,
        grid=(B, MEL // TM),
        in_specs=[
            pl.BlockSpec((1, L, D), lambda b, mt: (b, 0, 0)),
            pl.BlockSpec((1, 1, L), lambda b, mt: (b, 0, 0)),
            pl.BlockSpec((1, 1), const),
            pl.BlockSpec((D, F), const), pl.BlockSpec((D, F), const),
            pl.BlockSpec((D, F), const), pl.BlockSpec((1, F), const),
            pl.BlockSpec((1, F), const), pl.BlockSpec((1, F), const),
            pl.BlockSpec((F, F), const), pl.BlockSpec((F, F), const),
            pl.BlockSpec((F, F), const), pl.BlockSpec((1, F), const),
            pl.BlockSpec((1, F), const), pl.BlockSpec((1, F), const),
            pl.BlockSpec((1, F), const), pl.BlockSpec((1, 1), const),
        ],
        out_specs=[
            pl.BlockSpec((1, TM, L), lambda b, mt: (b, mt, 0)),
            pl.BlockSpec((1, 1, L), lambda b, mt: (b, 0, 0)),
        ],
        out_shape=[
            jax.ShapeDtypeStruct((B, MEL, L), jnp.float32),
            jax.ShapeDtypeStruct((B, 1, L), jnp.float32),
        ],
    )(x, t3, mml, w1p, w1c, w1n, b1, g1, be1,
      w2p, w2c, w2n, b2, g2, be2, lw, lb)
    return (out_flat.reshape(B, MEL, D), align, dp3.reshape(B, L))


# speedup vs baseline: 3.0287x; 3.0287x over previous
import jax
import jax.numpy as jnp
from jax import lax
from jax.experimental import pallas as pl
from jax.experimental.pallas import tpu as pltpu

MEL = 4096
TM = 512


def _layer_norm(h, g, b):
    mu = jnp.mean(h, axis=1, keepdims=True)
    var = jnp.mean((h - mu) ** 2, axis=1, keepdims=True)
    return (h - mu) / jnp.sqrt(var + 1e-5) * g + b


def _align_body(xbf_ref, t_ref, mml_ref, out_ref, al_ref, cs_ref, csp_ref):
    B = t_ref.shape[1]
    L = t_ref.shape[2]
    b = pl.program_id(0)
    mt = pl.program_id(1)

    @pl.when((b == 0) & (mt == 0))
    def _csum():
        dur_all = t_ref[0].astype(jnp.float32)
        tri = (lax.broadcasted_iota(jnp.int32, (L, L), 0)
               <= lax.broadcasted_iota(jnp.int32, (L, L), 1)).astype(jnp.float32)
        cs = jnp.dot(dur_all, tri, preferred_element_type=jnp.float32)
        cs_ref[...] = cs
        csp_ref[...] = cs - dur_all

    mv = mt * TM + lax.broadcasted_iota(jnp.int32, (TM, 1), 0)
    mvf = jnp.where(mv < mml_ref[0, 0], mv, -1).astype(jnp.float32)
    cs_b = cs_ref[pl.ds(b, 1), :]
    csp_b = csp_ref[pl.ds(b, 1), :]
    a = ((cs_b > mvf) & (csp_b <= mvf)).astype(jnp.float32)
    al_ref[0] = a
    out_ref[0] = jnp.dot(a.astype(jnp.bfloat16), xbf_ref[0],
                         preferred_element_type=jnp.float32)


def _dp_body(x_ref, w1p, w1c, w1n, b1, g1, be1,
             w2p, w2c, w2n, b2, g2, be2, lw, lb, dp_ref):
    D = x_ref.shape[2]
    L = x_ref.shape[1]
    xb = x_ref[0]
    zr = jnp.zeros((1, D), jnp.float32)
    xp = jnp.concatenate([zr, xb[:-1]], axis=0)
    xn = jnp.concatenate([xb[1:], zr], axis=0)
    h = (jnp.dot(xp, w1p[...], preferred_element_type=jnp.float32)
         + jnp.dot(xb, w1c[...], preferred_element_type=jnp.float32)
         + jnp.dot(xn, w1n[...], preferred_element_type=jnp.float32)
         + b1[...])
    h = jax.nn.relu(_layer_norm(h, g1[...], be1[...]))
    hp = jnp.concatenate([zr, h[:-1]], axis=0)
    hn = jnp.concatenate([h[1:], zr], axis=0)
    h2 = (jnp.dot(hp, w2p[...], preferred_element_type=jnp.float32)
          + jnp.dot(h, w2c[...], preferred_element_type=jnp.float32)
          + jnp.dot(hn, w2n[...], preferred_element_type=jnp.float32)
          + b2[...])
    h2 = jax.nn.relu(_layer_norm(h2, g2[...], be2[...]))
    dp = jnp.exp(jnp.sum(h2 * lw[...], axis=1) + lb[0, 0])
    dp_ref[0] = dp.reshape(1, L)


def kernel(x, target, mel_max_length,
           conv1_w, conv1_b, ln1_g, ln1_b,
           conv2_w, conv2_b, ln2_g, ln2_b,
           lin_w, lin_b):
    B, L, D = x.shape
    F = conv1_w.shape[0]
    t3 = target.reshape(1, B, L)
    mml = jnp.asarray(mel_max_length, jnp.int32).reshape(1, 1)
    xbf = x.astype(jnp.bfloat16)
    w1p = conv1_w[:, :, 0].T
    w1c = conv1_w[:, :, 1].T
    w1n = conv1_w[:, :, 2].T
    w2p = conv2_w[:, :, 0].T
    w2c = conv2_w[:, :, 1].T
    w2n = conv2_w[:, :, 2].T
    b1 = conv1_b.reshape(1, F)
    g1 = ln1_g.reshape(1, F)
    be1 = ln1_b.reshape(1, F)
    b2 = conv2_b.reshape(1, F)
    g2 = ln2_g.reshape(1, F)
    be2 = ln2_b.reshape(1, F)
    lw = lin_w.reshape(1, F)
    lb = lin_b.reshape(1, 1)

    const = lambda *_: (0, 0)
    out, align = pl.pallas_call(
        _align_body,
        grid=(B, MEL // TM),
        in_specs=[
            pl.BlockSpec((1, L, D), lambda b, mt: (b, 0, 0)),
            pl.BlockSpec((1, B, L), lambda b, mt: (0, 0, 0)),
            pl.BlockSpec((1, 1), const),
        ],
        out_specs=[
            pl.BlockSpec((1, TM, D), lambda b, mt: (b, mt, 0)),
            pl.BlockSpec((1, TM, L), lambda b, mt: (b, mt, 0)),
        ],
        out_shape=[
            jax.ShapeDtypeStruct((B, MEL, D), jnp.float32),
            jax.ShapeDtypeStruct((B, MEL, L), jnp.float32),
        ],
        scratch_shapes=[
            pltpu.VMEM((B, L), jnp.float32),
            pltpu.VMEM((B, L), jnp.float32),
        ],
    )(xbf, t3, mml)

    dp3 = pl.pallas_call(
        _dp_body,
        grid=(B,),
        in_specs=[
            pl.BlockSpec((1, L, D), lambda b: (b, 0, 0)),
            pl.BlockSpec((D, F), lambda b: (0, 0)), pl.BlockSpec((D, F), lambda b: (0, 0)),
            pl.BlockSpec((D, F), lambda b: (0, 0)), pl.BlockSpec((1, F), lambda b: (0, 0)),
            pl.BlockSpec((1, F), lambda b: (0, 0)), pl.BlockSpec((1, F), lambda b: (0, 0)),
            pl.BlockSpec((F, F), lambda b: (0, 0)), pl.BlockSpec((F, F), lambda b: (0, 0)),
            pl.BlockSpec((F, F), lambda b: (0, 0)), pl.BlockSpec((1, F), lambda b: (0, 0)),
            pl.BlockSpec((1, F), lambda b: (0, 0)), pl.BlockSpec((1, F), lambda b: (0, 0)),
            pl.BlockSpec((1, F), lambda b: (0, 0)), pl.BlockSpec((1, 1), lambda b: (0, 0)),
        ],
        out_specs=pl.BlockSpec((1, 1, L), lambda b: (b, 0, 0)),
        out_shape=jax.ShapeDtypeStruct((B, 1, L), jnp.float32),
    )(x, w1p, w1c, w1n, b1, g1, be1, w2p, w2c, w2n, b2, g2, be2, lw, lb)
    return (out, align, dp3.reshape(B, L))


# speedup vs baseline: 3.5225x; 1.1630x over previous
import jax
import jax.numpy as jnp
from jax import lax
from jax.experimental import pallas as pl
from jax.experimental.pallas import tpu as pltpu

MEL = 4096
TM = 1024


def _layer_norm(h, g, b):
    mu = jnp.mean(h, axis=1, keepdims=True)
    var = jnp.mean((h - mu) ** 2, axis=1, keepdims=True)
    return (h - mu) / jnp.sqrt(var + 1e-5) * g + b


def _align_body(xbf_ref, t_ref, mml_ref, out_ref, al_ref, cs_ref, csp_ref):
    B = t_ref.shape[1]
    L = t_ref.shape[2]
    b = pl.program_id(0)
    mt = pl.program_id(1)

    @pl.when((b == 0) & (mt == 0))
    def _csum():
        dur_all = t_ref[0].astype(jnp.float32)
        tri = (lax.broadcasted_iota(jnp.int32, (L, L), 0)
               <= lax.broadcasted_iota(jnp.int32, (L, L), 1)).astype(jnp.float32)
        cs = jnp.dot(dur_all, tri, preferred_element_type=jnp.float32)
        cs_ref[...] = cs
        csp_ref[...] = cs - dur_all

    mv = mt * TM + lax.broadcasted_iota(jnp.int32, (TM, 1), 0)
    mvf = jnp.where(mv < mml_ref[0, 0], mv, -1).astype(jnp.float32)
    cs_b = cs_ref[pl.ds(b, 1), :]
    csp_b = csp_ref[pl.ds(b, 1), :]
    a = ((cs_b > mvf) & (csp_b <= mvf)).astype(jnp.float32)
    al_ref[0] = a
    out_ref[0] = jnp.dot(a.astype(jnp.bfloat16), xbf_ref[0],
                         preferred_element_type=jnp.float32)


def _dp_body(x_ref, w1p, w1c, w1n, b1, g1, be1,
             w2p, w2c, w2n, b2, g2, be2, lw, lb, dp_ref):
    D = x_ref.shape[2]
    L = x_ref.shape[1]
    xb = x_ref[0]
    zr = jnp.zeros((1, D), jnp.float32)
    xp = jnp.concatenate([zr, xb[:-1]], axis=0)
    xn = jnp.concatenate([xb[1:], zr], axis=0)
    h = (jnp.dot(xp, w1p[...], preferred_element_type=jnp.float32)
         + jnp.dot(xb, w1c[...], preferred_element_type=jnp.float32)
         + jnp.dot(xn, w1n[...], preferred_element_type=jnp.float32)
         + b1[...])
    h = jax.nn.relu(_layer_norm(h, g1[...], be1[...]))
    hp = jnp.concatenate([zr, h[:-1]], axis=0)
    hn = jnp.concatenate([h[1:], zr], axis=0)
    h2 = (jnp.dot(hp, w2p[...], preferred_element_type=jnp.float32)
          + jnp.dot(h, w2c[...], preferred_element_type=jnp.float32)
          + jnp.dot(hn, w2n[...], preferred_element_type=jnp.float32)
          + b2[...])
    h2 = jax.nn.relu(_layer_norm(h2, g2[...], be2[...]))
    dp = jnp.exp(jnp.sum(h2 * lw[...], axis=1) + lb[0, 0])
    dp_ref[0] = dp.reshape(1, L)


def kernel(x, target, mel_max_length,
           conv1_w, conv1_b, ln1_g, ln1_b,
           conv2_w, conv2_b, ln2_g, ln2_b,
           lin_w, lin_b):
    B, L, D = x.shape
    F = conv1_w.shape[0]
    t3 = target.reshape(1, B, L)
    mml = jnp.asarray(mel_max_length, jnp.int32).reshape(1, 1)
    xbf = x.astype(jnp.bfloat16)
    w1p = conv1_w[:, :, 0].T
    w1c = conv1_w[:, :, 1].T
    w1n = conv1_w[:, :, 2].T
    w2p = conv2_w[:, :, 0].T
    w2c = conv2_w[:, :, 1].T
    w2n = conv2_w[:, :, 2].T
    b1 = conv1_b.reshape(1, F)
    g1 = ln1_g.reshape(1, F)
    be1 = ln1_b.reshape(1, F)
    b2 = conv2_b.reshape(1, F)
    g2 = ln2_g.reshape(1, F)
    be2 = ln2_b.reshape(1, F)
    lw = lin_w.reshape(1, F)
    lb = lin_b.reshape(1, 1)

    const = lambda *_: (0, 0)
    out, align = pl.pallas_call(
        _align_body,
        grid=(B, MEL // TM),
        in_specs=[
            pl.BlockSpec((1, L, D), lambda b, mt: (b, 0, 0)),
            pl.BlockSpec((1, B, L), lambda b, mt: (0, 0, 0)),
            pl.BlockSpec((1, 1), const),
        ],
        out_specs=[
            pl.BlockSpec((1, TM, D), lambda b, mt: (b, mt, 0)),
            pl.BlockSpec((1, TM, L), lambda b, mt: (b, mt, 0)),
        ],
        out_shape=[
            jax.ShapeDtypeStruct((B, MEL, D), jnp.float32),
            jax.ShapeDtypeStruct((B, MEL, L), jnp.float32),
        ],
        scratch_shapes=[
            pltpu.VMEM((B, L), jnp.float32),
            pltpu.VMEM((B, L), jnp.float32),
        ],
    )(xbf, t3, mml)

    dp3 = pl.pallas_call(
        _dp_body,
        grid=(B,),
        in_specs=[
            pl.BlockSpec((1, L, D), lambda b: (b, 0, 0)),
            pl.BlockSpec((D, F), lambda b: (0, 0)), pl.BlockSpec((D, F), lambda b: (0, 0)),
            pl.BlockSpec((D, F), lambda b: (0, 0)), pl.BlockSpec((1, F), lambda b: (0, 0)),
            pl.BlockSpec((1, F), lambda b: (0, 0)), pl.BlockSpec((1, F), lambda b: (0, 0)),
            pl.BlockSpec((F, F), lambda b: (0, 0)), pl.BlockSpec((F, F), lambda b: (0, 0)),
            pl.BlockSpec((F, F), lambda b: (0, 0)), pl.BlockSpec((1, F), lambda b: (0, 0)),
            pl.BlockSpec((1, F), lambda b: (0, 0)), pl.BlockSpec((1, F), lambda b: (0, 0)),
            pl.BlockSpec((1, F), lambda b: (0, 0)), pl.BlockSpec((1, 1), lambda b: (0, 0)),
        ],
        out_specs=pl.BlockSpec((1, 1, L), lambda b: (b, 0, 0)),
        out_shape=jax.ShapeDtypeStruct((B, 1, L), jnp.float32),
    )(x, w1p, w1c, w1n, b1, g1, be1, w2p, w2c, w2n, b2, g2, be2, lw, lb)
    return (out, align, dp3.reshape(B, L))


# speedup vs baseline: 4.1211x; 1.1699x over previous
import jax
import jax.numpy as jnp
from jax import lax
from jax.experimental import pallas as pl
from jax.experimental.pallas import tpu as pltpu

MEL = 4096
TM = 2048


def _layer_norm(h, g, b):
    mu = jnp.mean(h, axis=1, keepdims=True)
    var = jnp.mean((h - mu) ** 2, axis=1, keepdims=True)
    return (h - mu) / jnp.sqrt(var + 1e-5) * g + b


def _align_body(x_ref, t_ref, mml_ref, out_ref, al_ref, cs_ref, csp_ref):
    B = t_ref.shape[1]
    L = t_ref.shape[2]
    b = pl.program_id(0)
    mt = pl.program_id(1)

    @pl.when((b == 0) & (mt == 0))
    def _csum():
        dur_all = t_ref[0].astype(jnp.float32)
        tri = (lax.broadcasted_iota(jnp.int32, (L, L), 0)
               <= lax.broadcasted_iota(jnp.int32, (L, L), 1)).astype(jnp.float32)
        cs = jnp.dot(dur_all, tri, preferred_element_type=jnp.float32)
        cs_ref[...] = cs
        csp_ref[...] = cs - dur_all

    mv = mt * TM + lax.broadcasted_iota(jnp.int32, (TM, 1), 0)
    mvf = jnp.where(mv < mml_ref[0, 0], mv, -1).astype(jnp.float32)
    cs_b = cs_ref[pl.ds(b, 1), :]
    csp_b = csp_ref[pl.ds(b, 1), :]
    a = ((cs_b > mvf) & (csp_b <= mvf)).astype(jnp.float32)
    al_ref[0] = a
    out_ref[0] = jnp.dot(a.astype(jnp.bfloat16), x_ref[0].astype(jnp.bfloat16),
                         preferred_element_type=jnp.float32)


def _dp_body(x_ref, w1p, w1c, w1n, b1, g1, be1,
             w2p, w2c, w2n, b2, g2, be2, lw, lb, dp_ref):
    D = x_ref.shape[2]
    L = x_ref.shape[1]
    xb = x_ref[0]
    zr = jnp.zeros((1, D), jnp.float32)
    xp = jnp.concatenate([zr, xb[:-1]], axis=0)
    xn = jnp.concatenate([xb[1:], zr], axis=0)
    h = (jnp.dot(xp, w1p[...], preferred_element_type=jnp.float32)
         + jnp.dot(xb, w1c[...], preferred_element_type=jnp.float32)
         + jnp.dot(xn, w1n[...], preferred_element_type=jnp.float32)
         + b1[...])
    h = jax.nn.relu(_layer_norm(h, g1[...], be1[...]))
    hp = jnp.concatenate([zr, h[:-1]], axis=0)
    hn = jnp.concatenate([h[1:], zr], axis=0)
    h2 = (jnp.dot(hp, w2p[...], preferred_element_type=jnp.float32)
          + jnp.dot(h, w2c[...], preferred_element_type=jnp.float32)
          + jnp.dot(hn, w2n[...], preferred_element_type=jnp.float32)
          + b2[...])
    h2 = jax.nn.relu(_layer_norm(h2, g2[...], be2[...]))
    dp = jnp.exp(jnp.sum(h2 * lw[...], axis=1) + lb[0, 0])
    dp_ref[0] = dp.reshape(1, L)


def kernel(x, target, mel_max_length,
           conv1_w, conv1_b, ln1_g, ln1_b,
           conv2_w, conv2_b, ln2_g, ln2_b,
           lin_w, lin_b):
    B, L, D = x.shape
    F = conv1_w.shape[0]
    t3 = target.reshape(1, B, L)
    mml = jnp.asarray(mel_max_length, jnp.int32).reshape(1, 1)
    w1p = conv1_w[:, :, 0].T
    w1c = conv1_w[:, :, 1].T
    w1n = conv1_w[:, :, 2].T
    w2p = conv2_w[:, :, 0].T
    w2c = conv2_w[:, :, 1].T
    w2n = conv2_w[:, :, 2].T
    b1 = conv1_b.reshape(1, F)
    g1 = ln1_g.reshape(1, F)
    be1 = ln1_b.reshape(1, F)
    b2 = conv2_b.reshape(1, F)
    g2 = ln2_g.reshape(1, F)
    be2 = ln2_b.reshape(1, F)
    lw = lin_w.reshape(1, F)
    lb = lin_b.reshape(1, 1)

    const = lambda *_: (0, 0)
    out, align = pl.pallas_call(
        _align_body,
        grid=(B, MEL // TM),
        in_specs=[
            pl.BlockSpec((1, L, D), lambda b, mt: (b, 0, 0)),
            pl.BlockSpec((1, B, L), lambda b, mt: (0, 0, 0)),
            pl.BlockSpec((1, 1), const),
        ],
        out_specs=[
            pl.BlockSpec((1, TM, D), lambda b, mt: (b, mt, 0)),
            pl.BlockSpec((1, TM, L), lambda b, mt: (b, mt, 0)),
        ],
        out_shape=[
            jax.ShapeDtypeStruct((B, MEL, D), jnp.float32),
            jax.ShapeDtypeStruct((B, MEL, L), jnp.float32),
        ],
        scratch_shapes=[
            pltpu.VMEM((B, L), jnp.float32),
            pltpu.VMEM((B, L), jnp.float32),
        ],
    )(x, t3, mml)

    dp3 = pl.pallas_call(
        _dp_body,
        grid=(B,),
        in_specs=[
            pl.BlockSpec((1, L, D), lambda b: (b, 0, 0)),
            pl.BlockSpec((D, F), lambda b: (0, 0)), pl.BlockSpec((D, F), lambda b: (0, 0)),
            pl.BlockSpec((D, F), lambda b: (0, 0)), pl.BlockSpec((1, F), lambda b: (0, 0)),
            pl.BlockSpec((1, F), lambda b: (0, 0)), pl.BlockSpec((1, F), lambda b: (0, 0)),
            pl.BlockSpec((F, F), lambda b: (0, 0)), pl.BlockSpec((F, F), lambda b: (0, 0)),
            pl.BlockSpec((F, F), lambda b: (0, 0)), pl.BlockSpec((1, F), lambda b: (0, 0)),
            pl.BlockSpec((1, F), lambda b: (0, 0)), pl.BlockSpec((1, F), lambda b: (0, 0)),
            pl.BlockSpec((1, F), lambda b: (0, 0)), pl.BlockSpec((1, 1), lambda b: (0, 0)),
        ],
        out_specs=pl.BlockSpec((1, 1, L), lambda b: (b, 0, 0)),
        out_shape=jax.ShapeDtypeStruct((B, 1, L), jnp.float32),
    )(x, w1p, w1c, w1n, b1, g1, be1, w2p, w2c, w2n, b2, g2, be2, lw, lb)
    return (out, align, dp3.reshape(B, L))


# speedup vs baseline: 4.1390x; 1.0043x over previous
import jax
import jax.numpy as jnp
from jax import lax
from jax.experimental import pallas as pl
from jax.experimental.pallas import tpu as pltpu

MEL = 4096
TM = 4096


def _layer_norm(h, g, b):
    mu = jnp.mean(h, axis=1, keepdims=True)
    var = jnp.mean((h - mu) ** 2, axis=1, keepdims=True)
    return (h - mu) / jnp.sqrt(var + 1e-5) * g + b


def _align_body(x_ref, t_ref, mml_ref, out_ref, al_ref, cs_ref, csp_ref):
    B = t_ref.shape[1]
    L = t_ref.shape[2]
    b = pl.program_id(0)
    mt = pl.program_id(1)

    @pl.when((b == 0) & (mt == 0))
    def _csum():
        dur_all = t_ref[0].astype(jnp.float32)
        tri = (lax.broadcasted_iota(jnp.int32, (L, L), 0)
               <= lax.broadcasted_iota(jnp.int32, (L, L), 1)).astype(jnp.float32)
        cs = jnp.dot(dur_all, tri, preferred_element_type=jnp.float32)
        cs_ref[...] = cs
        csp_ref[...] = cs - dur_all

    mv = mt * TM + lax.broadcasted_iota(jnp.int32, (TM, 1), 0)
    mvf = jnp.where(mv < mml_ref[0, 0], mv, -1).astype(jnp.float32)
    cs_b = cs_ref[pl.ds(b, 1), :]
    csp_b = csp_ref[pl.ds(b, 1), :]
    a = ((cs_b > mvf) & (csp_b <= mvf)).astype(jnp.float32)
    al_ref[0] = a
    out_ref[0] = jnp.dot(a.astype(jnp.bfloat16), x_ref[0].astype(jnp.bfloat16),
                         preferred_element_type=jnp.float32)


def _dp_body(x_ref, w1p, w1c, w1n, b1, g1, be1,
             w2p, w2c, w2n, b2, g2, be2, lw, lb, dp_ref):
    D = x_ref.shape[2]
    L = x_ref.shape[1]
    xb = x_ref[0]
    zr = jnp.zeros((1, D), jnp.float32)
    xp = jnp.concatenate([zr, xb[:-1]], axis=0)
    xn = jnp.concatenate([xb[1:], zr], axis=0)
    h = (jnp.dot(xp, w1p[...], preferred_element_type=jnp.float32)
         + jnp.dot(xb, w1c[...], preferred_element_type=jnp.float32)
         + jnp.dot(xn, w1n[...], preferred_element_type=jnp.float32)
         + b1[...])
    h = jax.nn.relu(_layer_norm(h, g1[...], be1[...]))
    hp = jnp.concatenate([zr, h[:-1]], axis=0)
    hn = jnp.concatenate([h[1:], zr], axis=0)
    h2 = (jnp.dot(hp, w2p[...], preferred_element_type=jnp.float32)
          + jnp.dot(h, w2c[...], preferred_element_type=jnp.float32)
          + jnp.dot(hn, w2n[...], preferred_element_type=jnp.float32)
          + b2[...])
    h2 = jax.nn.relu(_layer_norm(h2, g2[...], be2[...]))
    dp = jnp.exp(jnp.sum(h2 * lw[...], axis=1) + lb[0, 0])
    dp_ref[0] = dp.reshape(1, L)


def kernel(x, target, mel_max_length,
           conv1_w, conv1_b, ln1_g, ln1_b,
           conv2_w, conv2_b, ln2_g, ln2_b,
           lin_w, lin_b):
    B, L, D = x.shape
    F = conv1_w.shape[0]
    t3 = target.reshape(1, B, L)
    mml = jnp.asarray(mel_max_length, jnp.int32).reshape(1, 1)
    w1p = conv1_w[:, :, 0].T
    w1c = conv1_w[:, :, 1].T
    w1n = conv1_w[:, :, 2].T
    w2p = conv2_w[:, :, 0].T
    w2c = conv2_w[:, :, 1].T
    w2n = conv2_w[:, :, 2].T
    b1 = conv1_b.reshape(1, F)
    g1 = ln1_g.reshape(1, F)
    be1 = ln1_b.reshape(1, F)
    b2 = conv2_b.reshape(1, F)
    g2 = ln2_g.reshape(1, F)
    be2 = ln2_b.reshape(1, F)
    lw = lin_w.reshape(1, F)
    lb = lin_b.reshape(1, 1)

    const = lambda *_: (0, 0)
    out, align = pl.pallas_call(
        _align_body,
        grid=(B, MEL // TM),
        in_specs=[
            pl.BlockSpec((1, L, D), lambda b, mt: (b, 0, 0)),
            pl.BlockSpec((1, B, L), lambda b, mt: (0, 0, 0)),
            pl.BlockSpec((1, 1), const),
        ],
        out_specs=[
            pl.BlockSpec((1, TM, D), lambda b, mt: (b, mt, 0)),
            pl.BlockSpec((1, TM, L), lambda b, mt: (b, mt, 0)),
        ],
        out_shape=[
            jax.ShapeDtypeStruct((B, MEL, D), jnp.float32),
            jax.ShapeDtypeStruct((B, MEL, L), jnp.float32),
        ],
        scratch_shapes=[
            pltpu.VMEM((B, L), jnp.float32),
            pltpu.VMEM((B, L), jnp.float32),
        ],
    )(x, t3, mml)

    dp3 = pl.pallas_call(
        _dp_body,
        grid=(B,),
        in_specs=[
            pl.BlockSpec((1, L, D), lambda b: (b, 0, 0)),
            pl.BlockSpec((D, F), lambda b: (0, 0)), pl.BlockSpec((D, F), lambda b: (0, 0)),
            pl.BlockSpec((D, F), lambda b: (0, 0)), pl.BlockSpec((1, F), lambda b: (0, 0)),
            pl.BlockSpec((1, F), lambda b: (0, 0)), pl.BlockSpec((1, F), lambda b: (0, 0)),
            pl.BlockSpec((F, F), lambda b: (0, 0)), pl.BlockSpec((F, F), lambda b: (0, 0)),
            pl.BlockSpec((F, F), lambda b: (0, 0)), pl.BlockSpec((1, F), lambda b: (0, 0)),
            pl.BlockSpec((1, F), lambda b: (0, 0)), pl.BlockSpec((1, F), lambda b: (0, 0)),
            pl.BlockSpec((1, F), lambda b: (0, 0)), pl.BlockSpec((1, 1), lambda b: (0, 0)),
        ],
        out_specs=pl.BlockSpec((1, 1, L), lambda b: (b, 0, 0)),
        out_shape=jax.ShapeDtypeStruct((B, 1, L), jnp.float32),
    )(x, w1p, w1c, w1n, b1, g1, be1, w2p, w2c, w2n, b2, g2, be2, lw, lb)
    return (out, align, dp3.reshape(B, L))
